# Initial kernel scaffold; baseline (speedup 1.0000x reference)
import jax
import jax.numpy as jnp
from jax.experimental import pallas as pl


def kernel(x_BLD, W_route, b_route, W_noise, b_noise, W1, b1, W2, b2):
    raise NotImplementedError("write your pallas kernel here")


# speedup vs baseline: 3.2396x; 3.2396x over previous
import functools

import jax
import jax.numpy as jnp
from jax.experimental import pallas as pl
from jax.experimental.pallas import tpu as pltpu

_NEG_INF = float("-inf")
_BIG = 1 << 20


def _router_body(x_ref, wr_ref, br_ref, wn_ref, bn_ref, nz_ref,
                 pos_ref, gate_ref, *, cap):
    x = x_ref[...]
    E = wr_ref.shape[1]
    F = x.shape[0]
    dn = (((0,), (1,)), ((), ()))
    logits = jax.lax.dot_general(wr_ref[...], x, dn,
                                 preferred_element_type=jnp.float32)
    logits = logits + br_ref[...]
    nlogits = jax.lax.dot_general(wn_ref[...], x, dn,
                                  preferred_element_type=jnp.float32)
    nlogits = nlogits + bn_ref[...]
    sp = jnp.log1p(jnp.exp(-jnp.abs(nlogits))) + jnp.maximum(nlogits, 0.0)
    noisy = logits + nz_ref[...] * sp

    e_iota = jax.lax.broadcasted_iota(jnp.int32, noisy.shape, 0)
    top1v = jnp.max(noisy, axis=0, keepdims=True)
    top1i = jnp.min(jnp.where(noisy == top1v, e_iota, E), axis=0,
                    keepdims=True)
    masked = jnp.where(e_iota == top1i, _NEG_INF, noisy)
    top2v = jnp.max(masked, axis=0, keepdims=True)
    top2i = jnp.min(jnp.where(masked == top2v, e_iota, E), axis=0,
                    keepdims=True)

    ed = jnp.exp(top2v - top1v)
    denom = 1.0 + ed
    g1 = 1.0 / denom
    g2 = ed / denom

    sel1 = e_iota == top1i
    sel2 = e_iota == top2i
    sel = jnp.logical_or(sel1, sel2)
    self32 = sel.astype(jnp.float32)

    r_iota = jax.lax.broadcasted_iota(jnp.int32, (F, F), 0)
    c_iota = jax.lax.broadcasted_iota(jnp.int32, (F, F), 1)
    ut = (r_iota <= c_iota).astype(jnp.float32)
    ranks = jnp.dot(self32, ut, preferred_element_type=jnp.float32)
    pos = jnp.where(sel, ranks.astype(jnp.int32) - 1, _BIG)

    gate = jnp.where(sel1, g1, jnp.where(sel2, g2, 0.0))
    gate = jnp.where(pos < cap, gate, 0.0)

    pos_ref[...] = pos
    gate_ref[...] = gate


def _ffn_body(pos_ref, gate_ref, x_ref, w1_ref, b1_ref, w2_ref, b2_ref,
              out_ref, p_scr, xg_scr, y_scr, *, cap, n_ffb):
    f = pl.program_id(1)
    e = pl.program_id(0)

    @pl.when(f == 0)
    def _build_gather():
        pos_row = pos_ref[0]
        slot_iota = jax.lax.broadcasted_iota(
            jnp.int32, (cap, pos_row.shape[1]), 0)
        p_scr[...] = (pos_row == slot_iota).astype(jnp.float32)
        xg_scr[...] = jnp.dot(p_scr[...], x_ref[...],
                              preferred_element_type=jnp.float32)

    h = jnp.dot(xg_scr[...], w1_ref[0], preferred_element_type=jnp.float32)
    h = h + b1_ref[0]
    h = h * 0.5 * (1.0 + jax.lax.erf(h * 0.7071067811865476))
    yb = jnp.dot(h, w2_ref[0], preferred_element_type=jnp.float32)

    @pl.when(f == 0)
    def _init_acc():
        y_scr[...] = yb

    @pl.when(f != 0)
    def _acc():
        y_scr[...] = y_scr[...] + yb

    @pl.when(f == n_ffb - 1)
    def _scatter():
        gate_slot = jnp.sum(p_scr[...] * gate_ref[0], axis=1,
                            keepdims=True)
        yg = (y_scr[...] + b2_ref[0]) * gate_slot
        contrib = jax.lax.dot_general(
            p_scr[...], yg, (((0,), (0,)), ((), ())),
            preferred_element_type=jnp.float32)

        @pl.when(e == 0)
        def _init_out():
            out_ref[...] = contrib

        @pl.when(e != 0)
        def _add_out():
            out_ref[...] = out_ref[...] + contrib


def kernel(x_BLD, W_route, b_route, W_noise, b_noise, W1, b1, W2, b2):
    Bs, Ls, Ds = x_BLD.shape
    F = Bs * Ls
    E = W_route.shape[1]
    FF = W1.shape[2]
    cap = int(F * 2 / E * 1.25)
    flat = x_BLD.reshape(F, Ds)
    noise_mat = jax.random.normal(jax.random.key(1234), (F, E),
                                  dtype=jnp.float32)

    pos, gate = pl.pallas_call(
        functools.partial(_router_body, cap=cap),
        out_shape=(
            jax.ShapeDtypeStruct((E, F), jnp.int32),
            jax.ShapeDtypeStruct((E, F), jnp.float32),
        ),
    )(flat, W_route, b_route.reshape(E, 1), W_noise, b_noise.reshape(E, 1),
      noise_mat.T)

    n_ffb = 4
    ffb = FF // n_ffb
    grid = (E, n_ffb)
    out = pl.pallas_call(
        functools.partial(_ffn_body, cap=cap, n_ffb=n_ffb),
        grid=grid,
        in_specs=[
            pl.BlockSpec((1, 1, F), lambda e, f: (e, 0, 0)),
            pl.BlockSpec((1, 1, F), lambda e, f: (e, 0, 0)),
            pl.BlockSpec((F, Ds), lambda e, f: (0, 0)),
            pl.BlockSpec((1, Ds, ffb), lambda e, f: (e, 0, f)),
            pl.BlockSpec((1, 1, ffb), lambda e, f: (e, 0, f)),
            pl.BlockSpec((1, ffb, Ds), lambda e, f: (e, f, 0)),
            pl.BlockSpec((1, 1, Ds), lambda e, f: (e, 0, 0)),
        ],
        out_specs=pl.BlockSpec((F, Ds), lambda e, f: (0, 0)),
        out_shape=jax.ShapeDtypeStruct((F, Ds), jnp.float32),
        scratch_shapes=[
            pltpu.VMEM((cap, F), jnp.float32),
            pltpu.VMEM((cap, Ds), jnp.float32),
            pltpu.VMEM((cap, Ds), jnp.float32),
        ],
    )(pos.reshape(E, 1, F), gate.reshape(E, 1, F), flat, W1,
      b1.reshape(E, 1, FF), W2, b2.reshape(E, 1, Ds))

    return out.reshape(Bs, Ls, Ds)
